# baseline (speedup 1.0000x reference)
import functools

import jax
import jax.numpy as jnp
import numpy as np
from jax import lax
from jax.experimental import pallas as pl
from jax.experimental.pallas import tpu as pltpu
from jax.experimental.pallas import tpu_sc as plsc

B, L, D, V = 4096, 200, 128, 10000
NC, NS = 2, 16
NW = NC * NS
RPW = B // NW
HL = L // 2
NH = 2 * RPW
NG = D // 32

def _enc_body(tok_hbm, w_hbm, out_hbm, tok_v, rows0_v, rows1_v, out_v,
              w_sh, sem0, sem1):
    wid = lax.axis_index("s") * NC + lax.axis_index("c")
    base_h = wid * NH

    @pl.when(lax.axis_index("s") == 0)
    def _():
        pltpu.sync_copy(w_hbm, w_sh)

    plsc.subcore_barrier()

    pltpu.sync_copy(tok_hbm.at[pl.ds(base_h, NH)], tok_v)

    def issue(r, rows_v, sem):
        pltpu.async_copy(w_sh.at[tok_v.at[2 * r]],
                         rows_v.at[pl.ds(0, HL)], sem)
        pltpu.async_copy(w_sh.at[tok_v.at[2 * r + 1]],
                         rows_v.at[pl.ds(HL, HL)], sem)

    def wait(rows_v, sem):
        pltpu.make_async_copy(w_sh.at[tok_v.at[0]],
                              rows_v.at[pl.ds(0, HL)], sem).wait()
        pltpu.make_async_copy(w_sh.at[tok_v.at[0]],
                              rows_v.at[pl.ds(HL, HL)], sem).wait()

    def accum_store(r, rows_v):
        def body(l, accs):
            new = []
            for g in range(NG):
                xi = rows_v[l, pl.ds(g * 16, 16)]
                lo = lax.bitcast_convert_type(
                    xi << jnp.int32(16), jnp.float32)
                hi = lax.bitcast_convert_type(xi, jnp.float32)
                new.append(accs[2 * g] + lo)
                new.append(accs[2 * g + 1] + hi)
            return tuple(new)

        accs = tuple(jnp.zeros((16,), jnp.float32) for _ in range(2 * NG))
        accs = lax.fori_loop(0, L, body, accs, unroll=4)
        for g in range(NG):
            out_v[r, pl.ds(g * 16, 16)] = accs[2 * g] * jnp.float32(1.0 / L)
            out_v[r, pl.ds(64 + g * 16, 16)] = (accs[2 * g + 1]
                                                * jnp.float32(1.0 / L))

    npair = RPW // 2
    issue(0, rows0_v, sem0)

    def pair_body(i, carry):
        r0 = 2 * i
        issue(r0 + 1, rows1_v, sem1)
        wait(rows0_v, sem0)
        accum_store(r0, rows0_v)

        @pl.when(i < npair - 1)
        def _():
            issue(r0 + 2, rows0_v, sem0)

        wait(rows1_v, sem1)
        accum_store(r0 + 1, rows1_v)
        return carry

    lax.fori_loop(0, npair, pair_body, 0)

    pltpu.sync_copy(out_v, out_hbm.at[pl.ds(wid * RPW, RPW)])


@jax.jit
def _encode(tok2, w_bf):
    mesh = plsc.VectorSubcoreMesh(core_axis_name="c", subcore_axis_name="s")
    f = functools.partial(
        pl.kernel,
        mesh=mesh,
        compiler_params=pltpu.CompilerParams(use_tc_tiling_on_sc=False),
        out_type=jax.ShapeDtypeStruct((B, D), jnp.float32),
        scratch_types=[
            pltpu.VMEM((NH, HL), jnp.int32),
            pltpu.VMEM((L, D // 2), jnp.int32),
            pltpu.VMEM((L, D // 2), jnp.int32),
            pltpu.VMEM((RPW, D), jnp.float32),
            pltpu.VMEM_SHARED((V, D // 2), jnp.int32),
            pltpu.SemaphoreType.DMA,
            pltpu.SemaphoreType.DMA,
        ],
    )(_enc_body)
    return f(tok2, w_bf)


def _pack_body(w_ref, out_ref):
    u = lax.bitcast_convert_type(w_ref[...], jnp.uint32)
    te = u[:, :D // 2]
    to = u[:, D // 2:]
    t = (te + jnp.uint32(0x7FFF) + ((te >> 16) & jnp.uint32(1))) >> 16
    h = jnp.where(to >= t + jnp.uint32(32768),
                  (to - t + jnp.uint32(32768)) >> 16, to >> 16)
    out_ref[...] = lax.bitcast_convert_type((h << 16) | t, jnp.int32)


def _pack_table(W):
    return pl.pallas_call(
        _pack_body,
        grid=(5,),
        in_specs=[pl.BlockSpec((V // 5, D), lambda i: (i, 0))],
        out_specs=pl.BlockSpec((V // 5, D // 2), lambda i: (i, 0)),
        out_shape=jax.ShapeDtypeStruct((V, D // 2), jnp.int32),
    )(W)


def kernel(text_tokens, W):
    tok2 = text_tokens.astype(jnp.int32).reshape(2 * B, HL)
    return _encode(tok2, _pack_table(W))


# speedup vs baseline: 1.0029x; 1.0029x over previous
import functools

import jax
import jax.numpy as jnp
from jax import lax
from jax.experimental import pallas as pl
from jax.experimental.pallas import tpu as pltpu
from jax.experimental.pallas import tpu_sc as plsc

B, L, D, V = 4096, 200, 128, 10000
NC, NS = 2, 16
NW = NC * NS
RPW = B // NW
HL = L // 2
NH = 2 * RPW
NG = D // 32

def _enc_body(tok_hbm, w_hbm, out_hbm, tok_v, rows0_v, rows1_v, out_v,
              w_sh, sem0, sem1):
    wid = lax.axis_index("s") * NC + lax.axis_index("c")
    base_h = wid * NH

    @pl.when(lax.axis_index("s") == 0)
    def _():
        pltpu.sync_copy(w_hbm, w_sh)

    plsc.subcore_barrier()

    pltpu.sync_copy(tok_hbm.at[pl.ds(base_h, NH)], tok_v)

    def issue(r, rows_v, sem):
        pltpu.async_copy(w_sh.at[tok_v.at[2 * r]],
                         rows_v.at[pl.ds(0, HL)], sem)
        pltpu.async_copy(w_sh.at[tok_v.at[2 * r + 1]],
                         rows_v.at[pl.ds(HL, HL)], sem)

    def wait(rows_v, sem):
        pltpu.make_async_copy(w_sh.at[tok_v.at[0]],
                              rows_v.at[pl.ds(0, HL)], sem).wait()
        pltpu.make_async_copy(w_sh.at[tok_v.at[0]],
                              rows_v.at[pl.ds(HL, HL)], sem).wait()

    def accum_store(r, rows_v):
        def body(l, accs):
            new = []
            for g in range(NG):
                xi = rows_v[l, pl.ds(g * 16, 16)]
                lo = lax.bitcast_convert_type(
                    xi << jnp.int32(16), jnp.float32)
                hi = lax.bitcast_convert_type(xi, jnp.float32)
                new.append(accs[2 * g] + lo)
                new.append(accs[2 * g + 1] + hi)
            return tuple(new)

        accs = tuple(jnp.zeros((16,), jnp.float32) for _ in range(2 * NG))
        accs = lax.fori_loop(0, L, body, accs, unroll=2)
        for g in range(NG):
            out_v[r, pl.ds(g * 16, 16)] = accs[2 * g] * jnp.float32(1.0 / L)
            out_v[r, pl.ds(64 + g * 16, 16)] = (accs[2 * g + 1]
                                                * jnp.float32(1.0 / L))

    npair = RPW // 2
    issue(0, rows0_v, sem0)

    def pair_body(i, carry):
        r0 = 2 * i
        issue(r0 + 1, rows1_v, sem1)
        wait(rows0_v, sem0)
        accum_store(r0, rows0_v)

        @pl.when(i < npair - 1)
        def _():
            issue(r0 + 2, rows0_v, sem0)

        wait(rows1_v, sem1)
        accum_store(r0 + 1, rows1_v)
        return carry

    lax.fori_loop(0, npair, pair_body, 0)

    pltpu.sync_copy(out_v, out_hbm.at[pl.ds(wid * RPW, RPW)])


@jax.jit
def _encode(tok2, w_bf):
    mesh = plsc.VectorSubcoreMesh(core_axis_name="c", subcore_axis_name="s")
    f = functools.partial(
        pl.kernel,
        mesh=mesh,
        compiler_params=pltpu.CompilerParams(use_tc_tiling_on_sc=False),
        out_type=jax.ShapeDtypeStruct((B, D), jnp.float32),
        scratch_types=[
            pltpu.VMEM((NH, HL), jnp.int32),
            pltpu.VMEM((L, D // 2), jnp.int32),
            pltpu.VMEM((L, D // 2), jnp.int32),
            pltpu.VMEM((RPW, D), jnp.float32),
            pltpu.VMEM_SHARED((V, D // 2), jnp.int32),
            pltpu.SemaphoreType.DMA,
            pltpu.SemaphoreType.DMA,
        ],
    )(_enc_body)
    return f(tok2, w_bf)


def _pack_body(w_ref, out_ref):
    u = lax.bitcast_convert_type(w_ref[...], jnp.uint32)
    te = u[:, :D // 2]
    to = u[:, D // 2:]
    t = (te + jnp.uint32(0x7FFF) + ((te >> 16) & jnp.uint32(1))) >> 16
    h = jnp.where(to >= t + jnp.uint32(32768),
                  (to - t + jnp.uint32(32768)) >> 16, to >> 16)
    out_ref[...] = lax.bitcast_convert_type((h << 16) | t, jnp.int32)


def _pack_table(W):
    return pl.pallas_call(
        _pack_body,
        grid=(5,),
        in_specs=[pl.BlockSpec((V // 5, D), lambda i: (i, 0))],
        out_specs=pl.BlockSpec((V // 5, D // 2), lambda i: (i, 0)),
        out_shape=jax.ShapeDtypeStruct((V, D // 2), jnp.int32),
    )(W)


def kernel(text_tokens, W):
    tok2 = text_tokens.astype(jnp.int32).reshape(2 * B, HL)
    return _encode(tok2, _pack_table(W))
